# Initial kernel scaffold; baseline (speedup 1.0000x reference)
import jax
import jax.numpy as jnp
from jax.experimental import pallas as pl


def kernel(verts, edges, W0, b0, W1, b1):
    raise NotImplementedError("write your pallas kernel here")


# speedup vs baseline: 6.5666x; 6.5666x over previous
import functools

import jax
import jax.numpy as jnp
from jax import lax
from jax.experimental import pallas as pl
from jax.experimental.pallas import tpu as pltpu
from jax.experimental.pallas import tpu_sc as plsc

V = 10000
E = 320000
D = 128

NC = 2
NS = 16
L = 16
NW = NC * NS

V_PAD = 10240
CHUNK = 128
CHUNKS = 157
E_PAD = NS * CHUNKS * CHUNK
ROWS_PER_TILE = V_PAD // NS


DH = D // NC


def _sc_scatter_body(vw1_hbm, e0_hbm, e1_hbm, out_hbm,
                     e0_v, e1_v, rows_a, rows_b, acc, sem_a, sem_b):
    c = lax.axis_index("c")
    s = lax.axis_index("s")
    wid = s

    zvec = jnp.zeros((L,), jnp.float32)

    @pl.loop(0, CHUNK)
    def _zero_rows(r):
        @pl.loop(0, DH // L)
        def _zero_cols(k):
            rows_a[r, pl.ds(k * L, L)] = zvec

    base = s * ROWS_PER_TILE

    @pl.loop(0, ROWS_PER_TILE // CHUNK)
    def _zero_acc(t):
        pltpu.sync_copy(rows_a, acc.at[pl.ds(base + t * CHUNK, CHUNK)])

    pltpu.sync_copy(e0_hbm.at[wid], e0_v)
    pltpu.sync_copy(e1_hbm.at[wid], e1_v)

    plsc.subcore_barrier()

    vw1_c = vw1_hbm.at[c]

    @pl.loop(0, CHUNKS)
    def _edge_chunk(j):
        ga = pltpu.async_copy(vw1_c.at[e1_v.at[j]], rows_a, sem_a)
        gb = pltpu.async_copy(vw1_c.at[e0_v.at[j]], rows_b, sem_b)
        ga.wait()
        pltpu.sync_copy(rows_a, acc.at[e0_v.at[j]], add=True)
        gb.wait()
        pltpu.sync_copy(rows_b, acc.at[e1_v.at[j]], add=True)

    plsc.subcore_barrier()

    pltpu.sync_copy(acc.at[pl.ds(base, ROWS_PER_TILE)],
                    out_hbm.at[c, pl.ds(base, ROWS_PER_TILE)])


_sc_scatter = pl.kernel(
    _sc_scatter_body,
    out_type=jax.ShapeDtypeStruct((NC, V_PAD, DH), jnp.float32),
    mesh=plsc.VectorSubcoreMesh(
        core_axis_name="c", subcore_axis_name="s",
        num_cores=NC, num_subcores=NS),
    scratch_types=[
        pltpu.VMEM((CHUNKS, CHUNK), jnp.int32),
        pltpu.VMEM((CHUNKS, CHUNK), jnp.int32),
        pltpu.VMEM((CHUNK, DH), jnp.float32),
        pltpu.VMEM((CHUNK, DH), jnp.float32),
        pltpu.VMEM_SHARED((V_PAD, DH), jnp.float32),
        pltpu.SemaphoreType.DMA,
        pltpu.SemaphoreType.DMA,
    ],
    compiler_params=pltpu.CompilerParams(use_tc_tiling_on_sc=False),
)


def _mm_body(x_ref, w_ref, b_ref, o_ref):
    o_ref[...] = lax.dot_general(
        x_ref[...], w_ref[...], (((1,), (1,)), ((), ())),
        preferred_element_type=jnp.float32) + b_ref[...]


def _combine_body(x_ref, w_ref, b_ref, p0_ref, p1_ref, o_ref):
    nsum = jnp.concatenate((p0_ref[0], p1_ref[0]), axis=-1)
    o_ref[...] = (lax.dot_general(
        x_ref[...], w_ref[...], (((1,), (1,)), ((), ())),
        preferred_element_type=jnp.float32)
        + b_ref[...] + nsum)


def kernel(verts, edges, W0, b0, W1, b1):
    verts_pad = jnp.zeros((V_PAD, D), jnp.float32).at[:V].set(verts)
    b0r = b0.reshape(1, D)
    b1r = b1.reshape(1, D)

    mm_rows = V_PAD // 8
    vw1 = pl.pallas_call(
        _mm_body,
        grid=(8,),
        in_specs=[
            pl.BlockSpec((mm_rows, D), lambda i: (i, 0)),
            pl.BlockSpec((D, D), lambda i: (0, 0)),
            pl.BlockSpec((1, D), lambda i: (0, 0)),
        ],
        out_specs=pl.BlockSpec((mm_rows, D), lambda i: (i, 0)),
        out_shape=jax.ShapeDtypeStruct((V_PAD, D), jnp.float32),
    )(verts_pad, W1, b1r)

    vw1_split = jnp.swapaxes(vw1.reshape(V_PAD, NC, DH), 0, 1)

    fill = jnp.full((E_PAD - E,), V, jnp.int32)
    e0 = jnp.concatenate([edges[:, 0], fill]).reshape(NS, CHUNKS, CHUNK)
    e1 = jnp.concatenate([edges[:, 1], fill]).reshape(NS, CHUNKS, CHUNK)

    partials = _sc_scatter(vw1_split, e0, e1)

    cb_rows = 400
    out = pl.pallas_call(
        _combine_body,
        grid=(V // cb_rows,),
        in_specs=[
            pl.BlockSpec((cb_rows, D), lambda i: (i, 0)),
            pl.BlockSpec((D, D), lambda i: (0, 0)),
            pl.BlockSpec((1, D), lambda i: (0, 0)),
            pl.BlockSpec((1, cb_rows, DH), lambda i: (0, i, 0)),
            pl.BlockSpec((1, cb_rows, DH), lambda i: (1, i, 0)),
        ],
        out_specs=pl.BlockSpec((cb_rows, D), lambda i: (i, 0)),
        out_shape=jax.ShapeDtypeStruct((V, D), jnp.float32),
    )(verts, W0, b0r, partials, partials)

    return out


# speedup vs baseline: 6.9847x; 1.0637x over previous
import functools

import jax
import jax.numpy as jnp
from jax import lax
from jax.experimental import pallas as pl
from jax.experimental.pallas import tpu as pltpu
from jax.experimental.pallas import tpu_sc as plsc

V = 10000
E = 320000
D = 128

NC = 2
NS = 16
L = 16
NW = NC * NS

V_PAD = 10240
CHUNK = 128
CHUNKS = 158
E_PAD = NS * CHUNKS * CHUNK
ROWS_PER_TILE = V_PAD // NS


DH = D // NC


def _sc_scatter_body(vw1_hbm, e0_hbm, e1_hbm, out_hbm,
                     e0_v, e1_v, a0, a1, b0, b1, acc,
                     gsa0, gsa1, gsb0, gsb1, ssa0, ssa1, ssb0, ssb1):
    c = lax.axis_index("c")
    s = lax.axis_index("s")

    A = (a0, a1)
    B = (b0, b1)
    GSA = (gsa0, gsa1)
    GSB = (gsb0, gsb1)
    SSA = (ssa0, ssa1)
    SSB = (ssb0, ssb1)

    zvec = jnp.zeros((L,), jnp.float32)

    @pl.loop(0, CHUNK)
    def _zero_rows(r):
        @pl.loop(0, DH // L)
        def _zero_cols(k):
            a0[r, pl.ds(k * L, L)] = zvec

    base = s * ROWS_PER_TILE

    @pl.loop(0, ROWS_PER_TILE // CHUNK)
    def _zero_acc(t):
        pltpu.sync_copy(a0, acc.at[pl.ds(base + t * CHUNK, CHUNK)])

    pltpu.sync_copy(e0_hbm.at[s], e0_v)
    pltpu.sync_copy(e1_hbm.at[s], e1_v)

    plsc.subcore_barrier()

    vw1_c = vw1_hbm.at[c]

    def fire_gathers(p, j):
        pltpu.async_copy(vw1_c.at[e1_v.at[j]], A[p], GSA[p])
        pltpu.async_copy(vw1_c.at[e0_v.at[j]], B[p], GSB[p])

    def wait_gathers(p, j):
        pltpu.make_async_copy(vw1_c.at[e1_v.at[j]], A[p], GSA[p]).wait()
        pltpu.make_async_copy(vw1_c.at[e0_v.at[j]], B[p], GSB[p]).wait()

    def wait_scatters(p, j):
        pltpu.make_async_copy(A[p], acc.at[e0_v.at[j]], SSA[p]).wait()
        pltpu.make_async_copy(B[p], acc.at[e1_v.at[j]], SSB[p]).wait()

    fire_gathers(0, 0)
    fire_gathers(1, 1)

    @pl.loop(0, CHUNKS, step=2)
    def _edge_chunk(j):
        for p in range(2):
            cur = j + p
            wait_gathers(p, cur)
            pltpu.async_copy(A[p], acc.at[e0_v.at[cur]], SSA[p], add=True)
            pltpu.async_copy(B[p], acc.at[e1_v.at[cur]], SSB[p], add=True)

            @pl.when(cur < CHUNKS - 2)
            def _prefetch():
                wait_scatters(p, cur)
                fire_gathers(p, cur + 2)

    wait_scatters(0, CHUNKS - 2)
    wait_scatters(1, CHUNKS - 1)

    plsc.subcore_barrier()

    pltpu.sync_copy(acc.at[pl.ds(base, ROWS_PER_TILE)],
                    out_hbm.at[c, pl.ds(base, ROWS_PER_TILE)])


_sc_scatter = pl.kernel(
    _sc_scatter_body,
    out_type=jax.ShapeDtypeStruct((NC, V_PAD, DH), jnp.float32),
    mesh=plsc.VectorSubcoreMesh(
        core_axis_name="c", subcore_axis_name="s",
        num_cores=NC, num_subcores=NS),
    scratch_types=[
        pltpu.VMEM((CHUNKS, CHUNK), jnp.int32),
        pltpu.VMEM((CHUNKS, CHUNK), jnp.int32),
        pltpu.VMEM((CHUNK, DH), jnp.float32),
        pltpu.VMEM((CHUNK, DH), jnp.float32),
        pltpu.VMEM((CHUNK, DH), jnp.float32),
        pltpu.VMEM((CHUNK, DH), jnp.float32),
        pltpu.VMEM_SHARED((V_PAD, DH), jnp.float32),
    ] + [pltpu.SemaphoreType.DMA] * 8,
    compiler_params=pltpu.CompilerParams(use_tc_tiling_on_sc=False),
)


def _mm_body(x_ref, w_ref, b_ref, o_ref):
    o_ref[...] = lax.dot_general(
        x_ref[...], w_ref[...], (((1,), (1,)), ((), ())),
        preferred_element_type=jnp.float32) + b_ref[...]


def _combine_body(x_ref, w_ref, b_ref, p0_ref, p1_ref, o_ref):
    nsum = jnp.concatenate((p0_ref[0], p1_ref[0]), axis=-1)
    o_ref[...] = (lax.dot_general(
        x_ref[...], w_ref[...], (((1,), (1,)), ((), ())),
        preferred_element_type=jnp.float32)
        + b_ref[...] + nsum)


def kernel(verts, edges, W0, b0, W1, b1):
    verts_pad = jnp.zeros((V_PAD, D), jnp.float32).at[:V].set(verts)
    b0r = b0.reshape(1, D)
    b1r = b1.reshape(1, D)

    mm_rows = V_PAD // 8
    vw1 = pl.pallas_call(
        _mm_body,
        grid=(8,),
        in_specs=[
            pl.BlockSpec((mm_rows, D), lambda i: (i, 0)),
            pl.BlockSpec((D, D), lambda i: (0, 0)),
            pl.BlockSpec((1, D), lambda i: (0, 0)),
        ],
        out_specs=pl.BlockSpec((mm_rows, D), lambda i: (i, 0)),
        out_shape=jax.ShapeDtypeStruct((V_PAD, D), jnp.float32),
    )(verts_pad, W1, b1r)

    vw1_split = jnp.swapaxes(vw1.reshape(V_PAD, NC, DH), 0, 1)

    fill = jnp.full((E_PAD - E,), V, jnp.int32)
    e0 = jnp.concatenate([edges[:, 0], fill]).reshape(NS, CHUNKS, CHUNK)
    e1 = jnp.concatenate([edges[:, 1], fill]).reshape(NS, CHUNKS, CHUNK)

    partials = _sc_scatter(vw1_split, e0, e1)

    cb_rows = 400
    out = pl.pallas_call(
        _combine_body,
        grid=(V // cb_rows,),
        in_specs=[
            pl.BlockSpec((cb_rows, D), lambda i: (i, 0)),
            pl.BlockSpec((D, D), lambda i: (0, 0)),
            pl.BlockSpec((1, D), lambda i: (0, 0)),
            pl.BlockSpec((1, cb_rows, DH), lambda i: (0, i, 0)),
            pl.BlockSpec((1, cb_rows, DH), lambda i: (1, i, 0)),
        ],
        out_specs=pl.BlockSpec((cb_rows, D), lambda i: (i, 0)),
        out_shape=jax.ShapeDtypeStruct((V, D), jnp.float32),
    )(verts, W0, b0r, partials, partials)

    return out


# speedup vs baseline: 8.4273x; 1.2065x over previous
import functools

import jax
import jax.numpy as jnp
from jax import lax
from jax.experimental import pallas as pl
from jax.experimental.pallas import tpu as pltpu
from jax.experimental.pallas import tpu_sc as plsc

V = 10000
E = 320000
D = 128

NC = 2
NS = 16
L = 16
NW = NC * NS

V_PAD = 10240
CHUNK = 128
CHUNKS = 160
E_PAD = NS * CHUNKS * CHUNK
ROWS_PER_TILE = V_PAD // NS


DH = D // NC


def _sc_scatter_body(vw1_hbm, e0_hbm, e1_hbm, out_hbm,
                     ie0, ie1, a0, a1, b0, b1, vw1_sp, acc,
                     gsa0, gsa1, gsb0, gsb1, ssa0, ssa1, ssb0, ssb1,
                     isem0, isem1, isem2, isem3):
    c = lax.axis_index("c")
    s = lax.axis_index("s")

    A = (a0, a1)
    B = (b0, b1)
    GSA = (gsa0, gsa1)
    GSB = (gsb0, gsb1)
    SSA = (ssa0, ssa1)
    SSB = (ssb0, ssb1)
    ISEM = (isem0, isem1, isem2, isem3)

    zvec = jnp.zeros((L,), jnp.float32)

    @pl.loop(0, CHUNK)
    def _zero_rows(r):
        @pl.loop(0, DH // L)
        def _zero_cols(k):
            a0[r, pl.ds(k * L, L)] = zvec

    base = s * ROWS_PER_TILE

    @pl.loop(0, ROWS_PER_TILE // CHUNK)
    def _zero_acc(t):
        pltpu.sync_copy(a0, acc.at[pl.ds(base + t * CHUNK, CHUNK)])

    pltpu.sync_copy(vw1_hbm.at[c, pl.ds(base, ROWS_PER_TILE)],
                    vw1_sp.at[pl.ds(base, ROWS_PER_TILE)])

    plsc.subcore_barrier()

    def fire_idx(x, slot):
        pltpu.async_copy(e0_hbm.at[s, x], ie0.at[slot], ISEM[slot])
        pltpu.async_copy(e1_hbm.at[s, x], ie1.at[slot], ISEM[slot])

    def wait_idx(x, slot):
        pltpu.make_async_copy(e0_hbm.at[s, x], ie0.at[slot], ISEM[slot]).wait()
        pltpu.make_async_copy(e1_hbm.at[s, x], ie1.at[slot], ISEM[slot]).wait()

    def fire_gathers(bp, slot):
        pltpu.async_copy(vw1_sp.at[ie1.at[slot]], A[bp], GSA[bp])
        pltpu.async_copy(vw1_sp.at[ie0.at[slot]], B[bp], GSB[bp])

    def wait_gathers(bp, slot):
        pltpu.make_async_copy(vw1_sp.at[ie1.at[slot]], A[bp], GSA[bp]).wait()
        pltpu.make_async_copy(vw1_sp.at[ie0.at[slot]], B[bp], GSB[bp]).wait()

    def fire_scatters(bp, slot):
        pltpu.async_copy(A[bp], acc.at[ie0.at[slot]], SSA[bp], add=True)
        pltpu.async_copy(B[bp], acc.at[ie1.at[slot]], SSB[bp], add=True)

    def wait_scatters(bp, slot):
        pltpu.make_async_copy(A[bp], acc.at[ie0.at[slot]], SSA[bp]).wait()
        pltpu.make_async_copy(B[bp], acc.at[ie1.at[slot]], SSB[bp]).wait()

    for x in range(4):
        fire_idx(x, x)
    wait_idx(0, 0)
    fire_gathers(0, 0)
    wait_idx(1, 1)
    fire_gathers(1, 1)

    @pl.loop(0, CHUNKS, step=4)
    def _edge_chunk(j):
        for p in range(4):
            cur = j + p
            bp = p & 1
            wait_gathers(bp, p)
            fire_scatters(bp, p)

            @pl.when(cur < CHUNKS - 2)
            def _prefetch():
                wait_scatters(bp, p)

                @pl.when(cur < CHUNKS - 4)
                def _refill():
                    fire_idx(cur + 4, p)

                wait_idx(cur + 2, (p + 2) & 3)
                fire_gathers(bp, (p + 2) & 3)

    wait_scatters(0, 2)
    wait_scatters(1, 3)

    plsc.subcore_barrier()

    pltpu.sync_copy(acc.at[pl.ds(base, ROWS_PER_TILE)],
                    out_hbm.at[c, pl.ds(base, ROWS_PER_TILE)])


_sc_scatter = pl.kernel(
    _sc_scatter_body,
    out_type=jax.ShapeDtypeStruct((NC, V_PAD, DH), jnp.float32),
    mesh=plsc.VectorSubcoreMesh(
        core_axis_name="c", subcore_axis_name="s",
        num_cores=NC, num_subcores=NS),
    scratch_types=[
        pltpu.VMEM((4, CHUNK), jnp.int32),
        pltpu.VMEM((4, CHUNK), jnp.int32),
        pltpu.VMEM((CHUNK, DH), jnp.float32),
        pltpu.VMEM((CHUNK, DH), jnp.float32),
        pltpu.VMEM((CHUNK, DH), jnp.float32),
        pltpu.VMEM((CHUNK, DH), jnp.float32),
        pltpu.VMEM_SHARED((V_PAD, DH), jnp.float32),
        pltpu.VMEM_SHARED((V_PAD, DH), jnp.float32),
    ] + [pltpu.SemaphoreType.DMA] * 12,
    compiler_params=pltpu.CompilerParams(use_tc_tiling_on_sc=False),
)


def _mm_body(x_ref, w_ref, b_ref, o_ref):
    o_ref[...] = lax.dot_general(
        x_ref[...], w_ref[...], (((1,), (1,)), ((), ())),
        preferred_element_type=jnp.float32) + b_ref[...]


def _combine_body(x_ref, w_ref, b_ref, p0_ref, p1_ref, o_ref):
    nsum = jnp.concatenate((p0_ref[0], p1_ref[0]), axis=-1)
    o_ref[...] = (lax.dot_general(
        x_ref[...], w_ref[...], (((1,), (1,)), ((), ())),
        preferred_element_type=jnp.float32)
        + b_ref[...] + nsum)


def kernel(verts, edges, W0, b0, W1, b1):
    verts_pad = jnp.zeros((V_PAD, D), jnp.float32).at[:V].set(verts)
    b0r = b0.reshape(1, D)
    b1r = b1.reshape(1, D)

    mm_rows = V_PAD // 8
    vw1 = pl.pallas_call(
        _mm_body,
        grid=(8,),
        in_specs=[
            pl.BlockSpec((mm_rows, D), lambda i: (i, 0)),
            pl.BlockSpec((D, D), lambda i: (0, 0)),
            pl.BlockSpec((1, D), lambda i: (0, 0)),
        ],
        out_specs=pl.BlockSpec((mm_rows, D), lambda i: (i, 0)),
        out_shape=jax.ShapeDtypeStruct((V_PAD, D), jnp.float32),
    )(verts_pad, W1, b1r)

    vw1_split = jnp.swapaxes(vw1.reshape(V_PAD, NC, DH), 0, 1)

    fill = jnp.full((E_PAD - E,), V, jnp.int32)
    e0 = jnp.concatenate([edges[:, 0], fill]).reshape(NS, CHUNKS, CHUNK)
    e1 = jnp.concatenate([edges[:, 1], fill]).reshape(NS, CHUNKS, CHUNK)

    partials = _sc_scatter(vw1_split, e0, e1)

    cb_rows = 400
    out = pl.pallas_call(
        _combine_body,
        grid=(V // cb_rows,),
        in_specs=[
            pl.BlockSpec((cb_rows, D), lambda i: (i, 0)),
            pl.BlockSpec((D, D), lambda i: (0, 0)),
            pl.BlockSpec((1, D), lambda i: (0, 0)),
            pl.BlockSpec((1, cb_rows, DH), lambda i: (0, i, 0)),
            pl.BlockSpec((1, cb_rows, DH), lambda i: (1, i, 0)),
        ],
        out_specs=pl.BlockSpec((cb_rows, D), lambda i: (i, 0)),
        out_shape=jax.ShapeDtypeStruct((V, D), jnp.float32),
    )(verts, W0, b0r, partials, partials)

    return out


# speedup vs baseline: 8.6610x; 1.0277x over previous
import functools

import jax
import jax.numpy as jnp
from jax import lax
from jax.experimental import pallas as pl
from jax.experimental.pallas import tpu as pltpu
from jax.experimental.pallas import tpu_sc as plsc

V = 10000
E = 320000
D = 128

NC = 2
NS = 16
L = 16
NW = NC * NS

V_PAD = 10240
CHUNK = 128
CHUNKS = 160
E_PAD = NS * CHUNKS * CHUNK
ROWS_PER_TILE = V_PAD // NS


DH = D // NC


def _sc_scatter_body(vw1_hbm, e0_hbm, e1_hbm, out_hbm,
                     ie0, ie1, a0, a1, b0, b1, vw1_sp, acc,
                     gsa0, gsa1, gsb0, gsb1, ssa0, ssa1, ssb0, ssb1,
                     isem0, isem1, isem2, isem3):
    c = lax.axis_index("c")
    s = lax.axis_index("s")

    A = (a0, a1)
    B = (b0, b1)
    GSA = (gsa0, gsa1)
    GSB = (gsb0, gsb1)
    SSA = (ssa0, ssa1)
    SSB = (ssb0, ssb1)
    ISEM = (isem0, isem1, isem2, isem3)

    zvec = jnp.zeros((L,), jnp.float32)

    @pl.loop(0, CHUNK)
    def _zero_rows(r):
        @pl.loop(0, DH // L)
        def _zero_cols(k):
            a0[r, pl.ds(k * L, L)] = zvec

    base = s * ROWS_PER_TILE

    @pl.loop(0, ROWS_PER_TILE // CHUNK)
    def _zero_acc(t):
        pltpu.sync_copy(a0, acc.at[pl.ds(base + t * CHUNK, CHUNK)])

    pltpu.sync_copy(vw1_hbm.at[c, pl.ds(base, ROWS_PER_TILE)],
                    vw1_sp.at[pl.ds(base, ROWS_PER_TILE)])

    plsc.subcore_barrier()

    def fire_idx(x, slot):
        pltpu.async_copy(e0_hbm.at[s, x], ie0.at[slot], ISEM[slot])
        pltpu.async_copy(e1_hbm.at[s, x], ie1.at[slot], ISEM[slot])

    def wait_idx(x, slot):
        pltpu.make_async_copy(e0_hbm.at[s, x], ie0.at[slot], ISEM[slot]).wait()
        pltpu.make_async_copy(e1_hbm.at[s, x], ie1.at[slot], ISEM[slot]).wait()

    def fire_gathers(bp, slot):
        pltpu.async_copy(vw1_sp.at[ie1.at[slot]], A[bp], GSA[bp])
        pltpu.async_copy(vw1_sp.at[ie0.at[slot]], B[bp], GSB[bp])

    def wait_gathers(bp, slot):
        pltpu.make_async_copy(vw1_sp.at[ie1.at[slot]], A[bp], GSA[bp]).wait()
        pltpu.make_async_copy(vw1_sp.at[ie0.at[slot]], B[bp], GSB[bp]).wait()

    def fire_scatters(bp, slot):
        pltpu.async_copy(A[bp], acc.at[ie0.at[slot]], SSA[bp], add=True)
        pltpu.async_copy(B[bp], acc.at[ie1.at[slot]], SSB[bp], add=True)

    def wait_scatters(bp, slot):
        pltpu.make_async_copy(A[bp], acc.at[ie0.at[slot]], SSA[bp]).wait()
        pltpu.make_async_copy(B[bp], acc.at[ie1.at[slot]], SSB[bp]).wait()

    for x in range(4):
        fire_idx(x, x)
    wait_idx(0, 0)
    fire_gathers(0, 0)
    wait_idx(1, 1)
    fire_gathers(1, 1)

    @pl.loop(0, CHUNKS, step=4)
    def _edge_chunk(j):
        for p in range(4):
            cur = j + p
            bp = p & 1
            wait_gathers(bp, p)
            fire_scatters(bp, p)

            @pl.when(cur < CHUNKS - 2)
            def _prefetch():
                wait_scatters(bp, p)

                @pl.when(cur < CHUNKS - 4)
                def _refill():
                    fire_idx(cur + 4, p)

                wait_idx(cur + 2, (p + 2) & 3)
                fire_gathers(bp, (p + 2) & 3)

    wait_scatters(0, 2)
    wait_scatters(1, 3)

    plsc.subcore_barrier()

    pltpu.sync_copy(acc.at[pl.ds(base, ROWS_PER_TILE)],
                    out_hbm.at[c, pl.ds(base, ROWS_PER_TILE)])


_sc_scatter = pl.kernel(
    _sc_scatter_body,
    out_type=jax.ShapeDtypeStruct((NC, V_PAD, DH), jnp.float32),
    mesh=plsc.VectorSubcoreMesh(
        core_axis_name="c", subcore_axis_name="s",
        num_cores=NC, num_subcores=NS),
    scratch_types=[
        pltpu.VMEM((4, CHUNK), jnp.int32),
        pltpu.VMEM((4, CHUNK), jnp.int32),
        pltpu.VMEM((CHUNK, DH), jnp.float32),
        pltpu.VMEM((CHUNK, DH), jnp.float32),
        pltpu.VMEM((CHUNK, DH), jnp.float32),
        pltpu.VMEM((CHUNK, DH), jnp.float32),
        pltpu.VMEM_SHARED((V_PAD, DH), jnp.float32),
        pltpu.VMEM_SHARED((V_PAD, DH), jnp.float32),
    ] + [pltpu.SemaphoreType.DMA] * 12,
    compiler_params=pltpu.CompilerParams(use_tc_tiling_on_sc=False),
)


def _mm_body(x_ref, w_ref, b_ref, o_ref):
    mm = lax.dot_general(
        x_ref[...], w_ref[...], (((1,), (1,)), ((), ())),
        preferred_element_type=jnp.float32) + b_ref[...]
    o_ref[0] = mm[:, :DH]
    o_ref[1] = mm[:, DH:]


def _combine_body(x_ref, w_ref, b_ref, p0_ref, p1_ref, o_ref):
    nsum = jnp.concatenate((p0_ref[0], p1_ref[0]), axis=-1)
    o_ref[...] = (lax.dot_general(
        x_ref[...], w_ref[...], (((1,), (1,)), ((), ())),
        preferred_element_type=jnp.float32)
        + b_ref[...] + nsum)


def kernel(verts, edges, W0, b0, W1, b1):
    b0r = b0.reshape(1, D)
    b1r = b1.reshape(1, D)

    mm_rows = V_PAD // 8
    vw1_split = pl.pallas_call(
        _mm_body,
        grid=(8,),
        in_specs=[
            pl.BlockSpec((mm_rows, D), lambda i: (i, 0)),
            pl.BlockSpec((D, D), lambda i: (0, 0)),
            pl.BlockSpec((1, D), lambda i: (0, 0)),
        ],
        out_specs=pl.BlockSpec((NC, mm_rows, DH), lambda i: (0, i, 0)),
        out_shape=jax.ShapeDtypeStruct((NC, V_PAD, DH), jnp.float32),
    )(verts, W1, b1r)

    fill = jnp.full((E_PAD - E,), V, jnp.int32)
    e0 = jnp.concatenate([edges[:, 0], fill]).reshape(NS, CHUNKS, CHUNK)
    e1 = jnp.concatenate([edges[:, 1], fill]).reshape(NS, CHUNKS, CHUNK)

    partials = _sc_scatter(vw1_split, e0, e1)

    cb_rows = 400
    out = pl.pallas_call(
        _combine_body,
        grid=(V // cb_rows,),
        in_specs=[
            pl.BlockSpec((cb_rows, D), lambda i: (i, 0)),
            pl.BlockSpec((D, D), lambda i: (0, 0)),
            pl.BlockSpec((1, D), lambda i: (0, 0)),
            pl.BlockSpec((1, cb_rows, DH), lambda i: (0, i, 0)),
            pl.BlockSpec((1, cb_rows, DH), lambda i: (1, i, 0)),
        ],
        out_specs=pl.BlockSpec((cb_rows, D), lambda i: (i, 0)),
        out_shape=jax.ShapeDtypeStruct((V, D), jnp.float32),
    )(verts, W0, b0r, partials, partials)

    return out


# speedup vs baseline: 10.3525x; 1.1953x over previous
import functools

import jax
import jax.numpy as jnp
from jax import lax
from jax.experimental import pallas as pl
from jax.experimental.pallas import tpu as pltpu
from jax.experimental.pallas import tpu_sc as plsc

V = 10000
E = 320000
D = 128

NC = 2
NS = 16
L = 16
NW = NC * NS

V_PAD = 10240
CHUNK = 128
CHUNKS = 320
E_PAD = NS * CHUNKS * CHUNK
ROWS_PER_TILE = V_PAD // NS


DH = D // NC


def _sc_scatter_body(vw1_hbm, dst_hbm, src_hbm, out_hbm,
                     idxd, idxs, rb0, rb1, rb2, rb3, vw1_sp, acc,
                     g0, g1, g2, g3, s0, s1, s2, s3,
                     i0, i1, i2, i3, i4, i5, i6, i7):
    c = lax.axis_index("c")
    s = lax.axis_index("s")

    RB = (rb0, rb1, rb2, rb3)
    GS = (g0, g1, g2, g3)
    SS = (s0, s1, s2, s3)
    IS = (i0, i1, i2, i3, i4, i5, i6, i7)

    zvec = jnp.zeros((L,), jnp.float32)

    @pl.loop(0, CHUNK)
    def _zero_rows(r):
        @pl.loop(0, DH // L)
        def _zero_cols(k):
            rb0[r, pl.ds(k * L, L)] = zvec

    base = s * ROWS_PER_TILE

    @pl.loop(0, ROWS_PER_TILE // CHUNK)
    def _zero_acc(t):
        pltpu.sync_copy(rb0, acc.at[pl.ds(base + t * CHUNK, CHUNK)])

    pltpu.sync_copy(vw1_hbm.at[c, pl.ds(base, ROWS_PER_TILE)],
                    vw1_sp.at[pl.ds(base, ROWS_PER_TILE)])

    plsc.subcore_barrier()

    def fire_idx(x, q):
        pltpu.async_copy(dst_hbm.at[s, x], idxd.at[q], IS[q])
        pltpu.async_copy(src_hbm.at[s, x], idxs.at[q], IS[q])

    def wait_idx(x, q):
        pltpu.make_async_copy(dst_hbm.at[s, x], idxd.at[q], IS[q]).wait()
        pltpu.make_async_copy(src_hbm.at[s, x], idxs.at[q], IS[q]).wait()

    def fire_gather(b, q):
        pltpu.async_copy(vw1_sp.at[idxs.at[q]], RB[b], GS[b])

    def wait_gather(b, q):
        pltpu.make_async_copy(vw1_sp.at[idxs.at[q]], RB[b], GS[b]).wait()

    def fire_scatter(b, q):
        pltpu.async_copy(RB[b], acc.at[idxd.at[q]], SS[b], add=True)

    def wait_scatter(b, q):
        pltpu.make_async_copy(RB[b], acc.at[idxd.at[q]], SS[b]).wait()

    for x in range(4):
        fire_idx(x, x)
    wait_idx(0, 0)
    fire_gather(0, 0)
    wait_idx(1, 1)
    fire_gather(1, 1)

    @pl.loop(0, CHUNKS, step=8)
    def _edge_chunk(j):
        for p in range(8):
            cur = j + p
            b = p & 3
            wait_gather(b, p)
            fire_scatter(b, p)

            @pl.when(cur < CHUNKS - 2)
            def _steady():
                @pl.when(cur >= 2)
                def _free_buf():
                    wait_scatter((b + 2) & 3, (p + 6) & 7)

                @pl.when(cur < CHUNKS - 4)
                def _refill():
                    fire_idx(cur + 4, (p + 4) & 7)

                wait_idx(cur + 2, (p + 2) & 7)
                fire_gather((b + 2) & 3, (p + 2) & 7)

    for p in range(4):
        wait_scatter(p, p)

    plsc.subcore_barrier()

    pltpu.sync_copy(acc.at[pl.ds(base, ROWS_PER_TILE)],
                    out_hbm.at[c, pl.ds(base, ROWS_PER_TILE)])


_sc_scatter = pl.kernel(
    _sc_scatter_body,
    out_type=jax.ShapeDtypeStruct((NC, V_PAD, DH), jnp.float32),
    mesh=plsc.VectorSubcoreMesh(
        core_axis_name="c", subcore_axis_name="s",
        num_cores=NC, num_subcores=NS),
    scratch_types=[
        pltpu.VMEM((8, CHUNK), jnp.int32),
        pltpu.VMEM((8, CHUNK), jnp.int32),
        pltpu.VMEM((CHUNK, DH), jnp.float32),
        pltpu.VMEM((CHUNK, DH), jnp.float32),
        pltpu.VMEM((CHUNK, DH), jnp.float32),
        pltpu.VMEM((CHUNK, DH), jnp.float32),
        pltpu.VMEM_SHARED((V_PAD, DH), jnp.float32),
        pltpu.VMEM_SHARED((V_PAD, DH), jnp.float32),
    ] + [pltpu.SemaphoreType.DMA] * 16,
    compiler_params=pltpu.CompilerParams(use_tc_tiling_on_sc=False),
)


def _mm_body(x_ref, w_ref, b_ref, o_ref):
    mm = lax.dot_general(
        x_ref[...], w_ref[...], (((1,), (1,)), ((), ())),
        preferred_element_type=jnp.float32) + b_ref[...]
    o_ref[0] = mm[:, :DH]
    o_ref[1] = mm[:, DH:]


def _combine_body(x_ref, w_ref, b_ref, p0_ref, p1_ref, o_ref):
    nsum = jnp.concatenate((p0_ref[0], p1_ref[0]), axis=-1)
    o_ref[...] = (lax.dot_general(
        x_ref[...], w_ref[...], (((1,), (1,)), ((), ())),
        preferred_element_type=jnp.float32)
        + b_ref[...] + nsum)


def kernel(verts, edges, W0, b0, W1, b1):
    b0r = b0.reshape(1, D)
    b1r = b1.reshape(1, D)

    mm_rows = V_PAD // 8
    vw1_split = pl.pallas_call(
        _mm_body,
        grid=(8,),
        in_specs=[
            pl.BlockSpec((mm_rows, D), lambda i: (i, 0)),
            pl.BlockSpec((D, D), lambda i: (0, 0)),
            pl.BlockSpec((1, D), lambda i: (0, 0)),
        ],
        out_specs=pl.BlockSpec((NC, mm_rows, DH), lambda i: (0, i, 0)),
        out_shape=jax.ShapeDtypeStruct((NC, V_PAD, DH), jnp.float32),
    )(verts, W1, b1r)

    fill = jnp.full((E_PAD - 2 * E,), V, jnp.int32)
    dst = jnp.concatenate([edges[:, 0], edges[:, 1], fill]).reshape(
        NS, CHUNKS, CHUNK)
    src = jnp.concatenate([edges[:, 1], edges[:, 0], fill]).reshape(
        NS, CHUNKS, CHUNK)

    partials = _sc_scatter(vw1_split, dst, src)

    cb_rows = 400
    out = pl.pallas_call(
        _combine_body,
        grid=(V // cb_rows,),
        in_specs=[
            pl.BlockSpec((cb_rows, D), lambda i: (i, 0)),
            pl.BlockSpec((D, D), lambda i: (0, 0)),
            pl.BlockSpec((1, D), lambda i: (0, 0)),
            pl.BlockSpec((1, cb_rows, DH), lambda i: (0, i, 0)),
            pl.BlockSpec((1, cb_rows, DH), lambda i: (1, i, 0)),
        ],
        out_specs=pl.BlockSpec((cb_rows, D), lambda i: (i, 0)),
        out_shape=jax.ShapeDtypeStruct((V, D), jnp.float32),
    )(verts, W0, b0r, partials, partials)

    return out
